# Initial kernel scaffold; baseline (speedup 1.0000x reference)
import jax
import jax.numpy as jnp
from jax.experimental import pallas as pl


def kernel(x, edge_index, batch):
    raise NotImplementedError("write your pallas kernel here")


# speedup vs baseline: 5.3990x; 5.3990x over previous
import functools

import jax
import jax.numpy as jnp
from jax import lax
from jax.experimental import pallas as pl
from jax.experimental.pallas import tpu as pltpu
from jax.experimental.pallas import tpu_sc as plsc

G = 64
L = 128
NC = 2
NS = 16
NW = NC * NS


def _build_group_ids(batch, n, ng):
    pad = ng * L - n
    if pad == 0:
        return batch.reshape(ng, L)
    head = batch[: (ng - 1) * L].reshape(ng - 1, L)
    last = jnp.concatenate(
        [jnp.full((pad,), G, jnp.int32), batch[n - (L - pad):]])
    return jnp.concatenate([head, last[None]], axis=0)


def _sc_partial(x, bidx, n, d, ng):
    mesh = plsc.VectorSubcoreMesh(
        core_axis_name="c", subcore_axis_name="s",
        num_cores=NC, num_subcores=NS)
    nfull, nrem = ng // NW, ng % NW

    @functools.partial(
        pl.kernel,
        out_type=jax.ShapeDtypeStruct((NC, G + 1, d), jnp.float32),
        mesh=mesh,
        scratch_types=[
            pltpu.VMEM((L, d), jnp.float32),
            pltpu.VMEM((L,), jnp.int32),
            pltpu.VMEM((G + 1, d), jnp.float32),
            pltpu.VMEM_SHARED((G + 1, d), jnp.float32),
        ],
    )
    def k(x_hbm, bidx_hbm, psum_hbm, xblk, idxv, outv, acc_sh):
        c = lax.axis_index("c")
        s = lax.axis_index("s")
        wid = c * NS + s

        zero16 = jnp.zeros((16,), jnp.float32)

        @pl.when(s == 0)
        def _():
            def zrow(r, carry):
                for f in range(d // 16):
                    outv[r, pl.ds(f * 16, 16)] = zero16
                return carry
            lax.fori_loop(0, G + 1, zrow, 0)
            pltpu.sync_copy(outv, acc_sh)
        plsc.subcore_barrier()

        ngw = jnp.where(wid < nrem, nfull + 1, nfull)

        def grp(kk, carry):
            g = wid + kk * NW
            base = jnp.minimum(g * L, n - L)
            pltpu.sync_copy(bidx_hbm.at[g], idxv)
            pltpu.sync_copy(x_hbm.at[pl.ds(base, L)], xblk)
            pltpu.sync_copy(xblk, acc_sh.at[idxv], add=True)
            return carry
        lax.fori_loop(0, ngw, grp, 0)
        plsc.subcore_barrier()

        @pl.when(s == 0)
        def _():
            pltpu.sync_copy(acc_sh, outv)
            pltpu.sync_copy(outv, psum_hbm.at[c])

    return k(x, bidx)


def _finalize(psum, bidx, d):
    def body(ps_ref, b_ref, o_ref):
        sums = ps_ref[0, :G, :] + ps_ref[1, :G, :]
        b2 = b_ref[...]
        per_lane = [jnp.sum(jnp.where(b2 == g, 1.0, 0.0), axis=0)
                    for g in range(G)]
        cnt = jnp.sum(jnp.stack(per_lane, axis=0), axis=1, keepdims=True)
        o_ref[...] = sums / jnp.maximum(cnt, 1.0)

    return pl.pallas_call(
        body,
        out_shape=jax.ShapeDtypeStruct((G, d), jnp.float32),
    )(psum, bidx)


def kernel(x, edge_index, batch):
    n, d = x.shape
    ng = (n + L - 1) // L
    bidx = _build_group_ids(batch, n, ng)
    psum = _sc_partial(x, bidx, n, d, ng)
    return _finalize(psum, bidx, d)


# speedup vs baseline: 5.6263x; 1.0421x over previous
import functools

import jax
import jax.numpy as jnp
from jax import lax
from jax.experimental import pallas as pl
from jax.experimental.pallas import tpu as pltpu
from jax.experimental.pallas import tpu_sc as plsc

G = 64
L = 128
S = 3
R = S * L
NC = 2
NS = 16
NW = NC * NS


def _build_group_ids(batch, n, nsg):
    pad = nsg * R - n
    if pad == 0:
        return batch.reshape(nsg, S, L)
    head = batch[: (nsg - 1) * R]
    last = jnp.concatenate(
        [jnp.full((pad,), G, jnp.int32), batch[n - (R - pad):]])
    return jnp.concatenate([head, last]).reshape(nsg, S, L)


def _sc_partial(x, bidx, n, d, nsg):
    mesh = plsc.VectorSubcoreMesh(
        core_axis_name="c", subcore_axis_name="s",
        num_cores=NC, num_subcores=NS)
    nfull, nrem = nsg // NW, nsg % NW
    nit = nfull + (1 if nrem else 0)

    @functools.partial(
        pl.kernel,
        out_type=jax.ShapeDtypeStruct((NC, G + 1, d), jnp.float32),
        mesh=mesh,
        scratch_types=[
            pltpu.VMEM((R, d), jnp.float32),
            pltpu.VMEM((R, d), jnp.float32),
            pltpu.VMEM((S, L), jnp.int32),
            pltpu.VMEM((S, L), jnp.int32),
            pltpu.VMEM((G + 1, d), jnp.float32),
            pltpu.VMEM_SHARED((G + 1, d), jnp.float32),
            pltpu.SemaphoreType.DMA,
            pltpu.SemaphoreType.DMA,
            pltpu.SemaphoreType.DMA,
            pltpu.SemaphoreType.DMA,
            pltpu.SemaphoreType.DMA,
            pltpu.SemaphoreType.DMA,
        ],
    )
    def k(x_hbm, bidx_hbm, psum_hbm,
          xbuf0, xbuf1, ibuf0, ibuf1, outv, acc_sh,
          semx0, semx1, semi0, semi1, sems0, sems1):
        c = lax.axis_index("c")
        s = lax.axis_index("s")
        wid = c * NS + s
        xbufs, ibufs = (xbuf0, xbuf1), (ibuf0, ibuf1)
        semx, semi, sems = (semx0, semx1), (semi0, semi1), (sems0, sems1)

        def load_descs(kk):
            b = kk & 1
            t = wid + kk * NW
            base = jnp.minimum(t * R, n - R)
            return (pltpu.make_async_copy(
                        x_hbm.at[pl.ds(base, R)], xbufs[b], semx[b]),
                    pltpu.make_async_copy(bidx_hbm.at[t], ibufs[b], semi[b]))

        def issue_loads(kk):
            for dsc in load_descs(kk):
                dsc.start()

        def wait_loads(kk):
            for dsc in load_descs(kk):
                dsc.wait()

        def scatter_descs(kk):
            b = kk & 1
            return [pltpu.make_async_copy(
                        xbufs[b].at[pl.ds(j * L, L)],
                        acc_sh.at[ibufs[b].at[j]], sems[b])
                    for j in range(S)]

        def issue_scatters(kk):
            for dsc in scatter_descs(kk):
                dsc.start(add=True)

        def drain_scatters(kk):
            for dsc in scatter_descs(kk):
                dsc.wait()

        issue_loads(0)

        zero16 = jnp.zeros((16,), jnp.float32)

        @pl.when(s == 0)
        def _():
            def zrow(r, carry):
                for f in range(d // 16):
                    outv[r, pl.ds(f * 16, 16)] = zero16
                return carry
            lax.fori_loop(0, G + 1, zrow, 0)
            pltpu.sync_copy(outv, acc_sh)
        plsc.subcore_barrier()

        for kk in range(nit):
            guarded = nrem and kk == nfull
            if kk >= 1:
                drain_scatters(kk - 1)
            if guarded:
                @pl.when(wid < nrem)
                def _(kk=kk):
                    wait_loads(kk)
                    issue_scatters(kk)
            else:
                wait_loads(kk)
                issue_scatters(kk)
                if kk + 1 < nit:
                    if nrem and kk + 1 == nfull:
                        @pl.when(wid < nrem)
                        def _(kk=kk):
                            issue_loads(kk + 1)
                    else:
                        issue_loads(kk + 1)
        if nrem:
            @pl.when(wid < nrem)
            def _():
                drain_scatters(nit - 1)
        else:
            drain_scatters(nit - 1)
        plsc.subcore_barrier()

        @pl.when(s == 0)
        def _():
            pltpu.sync_copy(acc_sh, outv)
            pltpu.sync_copy(outv, psum_hbm.at[c])

    return k(x, bidx)


def _finalize(psum, bidx, d):
    def body(ps_ref, b_ref, o_ref):
        sums = ps_ref[0, :G, :] + ps_ref[1, :G, :]
        b2 = b_ref[...].reshape(-1, L)
        per_lane = [jnp.sum(jnp.where(b2 == g, 1.0, 0.0), axis=0)
                    for g in range(G)]
        cnt = jnp.sum(jnp.stack(per_lane, axis=0), axis=1, keepdims=True)
        o_ref[...] = sums / jnp.maximum(cnt, 1.0)

    return pl.pallas_call(
        body,
        out_shape=jax.ShapeDtypeStruct((G, d), jnp.float32),
    )(psum, bidx)


def kernel(x, edge_index, batch):
    n, d = x.shape
    nsg = (n + R - 1) // R
    bidx = _build_group_ids(batch, n, nsg)
    psum = _sc_partial(x, bidx, n, d, nsg)
    return _finalize(psum, bidx, d)


# speedup vs baseline: 8.8582x; 1.5744x over previous
import functools

import jax
import jax.numpy as jnp
from jax import lax
from jax.experimental import pallas as pl
from jax.experimental.pallas import tpu as pltpu
from jax.experimental.pallas import tpu_sc as plsc

G = 64
GA = 80
L = 128
S = 3
R = S * L
NC = 2
NS = 16
NW = NC * NS


def _sc_partial(x, batch, n, d, nsg):
    mesh = plsc.VectorSubcoreMesh(
        core_axis_name="c", subcore_axis_name="s",
        num_cores=NC, num_subcores=NS)
    nfull, nrem = nsg // NW, nsg % NW
    nit = nfull + (1 if nrem else 0)
    pad = nsg * R - n
    assert pad % 16 == 0 and pad < R and nfull >= 1
    last_w, last_k = (nsg - 1) % NW, (nsg - 1) // NW

    @functools.partial(
        pl.kernel,
        out_type=jax.ShapeDtypeStruct((NC, GA, d), jnp.float32),
        mesh=mesh,
        scratch_types=[
            pltpu.VMEM((R, d), jnp.float32),
            pltpu.VMEM((R, d), jnp.float32),
            pltpu.VMEM((S, L), jnp.int32),
            pltpu.VMEM((S, L), jnp.int32),
            pltpu.VMEM((GA, d), jnp.float32),
            pltpu.VMEM_SHARED((GA, d), jnp.float32),
            pltpu.SemaphoreType.DMA,
            pltpu.SemaphoreType.DMA,
            pltpu.SemaphoreType.DMA,
            pltpu.SemaphoreType.DMA,
            pltpu.SemaphoreType.DMA,
            pltpu.SemaphoreType.DMA,
        ],
    )
    def k(x_hbm, b_hbm, psum_hbm,
          xbuf0, xbuf1, ibuf0, ibuf1, outv, acc_sh,
          semx0, semx1, semi0, semi1, sems0, sems1):
        c = lax.axis_index("c")
        s = lax.axis_index("s")
        wid = c * NS + s
        xbufs, ibufs = (xbuf0, xbuf1), (ibuf0, ibuf1)
        semx, semi, sems = (semx0, semx1), (semi0, semi1), (sems0, sems1)

        def load_descs(kk):
            b = kk & 1
            t = wid + kk * NW
            base = jnp.minimum(t * R, n - R)
            descs = [pltpu.make_async_copy(
                x_hbm.at[pl.ds(base, R)], xbufs[b], semx[b])]
            for j in range(S):
                descs.append(pltpu.make_async_copy(
                    b_hbm.at[pl.ds(base + j * L, L)], ibufs[b].at[j], semi[b]))
            return descs

        def issue_loads(kk):
            for dsc in load_descs(kk):
                dsc.start()

        def wait_loads(kk):
            for dsc in load_descs(kk):
                dsc.wait()

        def scatter_descs(kk):
            b = kk & 1
            return [pltpu.make_async_copy(
                        xbufs[b].at[pl.ds(j * L, L)],
                        acc_sh.at[ibufs[b].at[j]], sems[b])
                    for j in range(S)]

        zero16 = jnp.zeros((16,), jnp.float32)
        dummy16 = jnp.full((16,), G, jnp.int32)

        def fix_pad(kk):
            b = kk & 1
            if pad and kk == last_k:
                @pl.when(wid == last_w)
                def _():
                    for e in range(pad // 16):
                        ibufs[b][e // (L // 16),
                                 pl.ds((e % (L // 16)) * 16, 16)] = dummy16

        issue_loads(0)

        @pl.when(s == 0)
        def _():
            def zrow(r_, carry):
                for f in range(d // 16):
                    outv[r_, pl.ds(f * 16, 16)] = zero16
                return carry
            lax.fori_loop(0, GA, zrow, 0)
            pltpu.sync_copy(outv, acc_sh)
        plsc.subcore_barrier()

        for kk in range(nit):
            guarded = nrem and kk == nfull
            if kk >= 1:
                for dsc in scatter_descs(kk - 1):
                    dsc.wait()
            if guarded:
                @pl.when(wid < nrem)
                def _(kk=kk):
                    wait_loads(kk)
                    fix_pad(kk)
                    for dsc in scatter_descs(kk):
                        dsc.start(add=True)
            else:
                wait_loads(kk)
                fix_pad(kk)
                for dsc in scatter_descs(kk):
                    dsc.start(add=True)
                if kk + 1 < nit:
                    if nrem and kk + 1 == nfull:
                        @pl.when(wid < nrem)
                        def _(kk=kk):
                            issue_loads(kk + 1)
                    else:
                        issue_loads(kk + 1)
        if nrem:
            @pl.when(wid < nrem)
            def _():
                for dsc in scatter_descs(nit - 1):
                    dsc.wait()
        else:
            for dsc in scatter_descs(nit - 1):
                dsc.wait()
        plsc.subcore_barrier()

        @pl.when(s == 0)
        def _():
            pltpu.sync_copy(acc_sh, outv)
            pltpu.sync_copy(outv, psum_hbm.at[c])

    return k(x, batch)


def _count_tc(bp):
    def body(b_ref, o_ref):
        b2 = b_ref[...]
        per_lane = [jnp.sum(jnp.where(b2 == g, 1.0, 0.0), axis=0)
                    for g in range(G)]
        o_ref[...] = jnp.sum(jnp.stack(per_lane, axis=0), axis=1,
                             keepdims=True)

    return pl.pallas_call(
        body,
        out_shape=jax.ShapeDtypeStruct((G, 1), jnp.float32),
    )(bp)


def _finalize(psum, cnt, d):
    def body(ps_ref, c_ref, o_ref):
        sums = ps_ref[0, :G, :] + ps_ref[1, :G, :]
        o_ref[...] = sums / jnp.maximum(c_ref[...], 1.0)

    return pl.pallas_call(
        body,
        out_shape=jax.ShapeDtypeStruct((G, d), jnp.float32),
    )(psum, cnt)


def kernel(x, edge_index, batch):
    n, d = x.shape
    nsg = (n + R - 1) // R
    ng = (n + L - 1) // L
    bp = jnp.concatenate(
        [batch, jnp.full((ng * L - n,), G, jnp.int32)]).reshape(ng, L)
    psum = _sc_partial(x, batch, n, d, nsg)
    cnt = _count_tc(bp)
    return _finalize(psum, cnt, d)
